# Initial kernel scaffold; baseline (speedup 1.0000x reference)
import functools

import jax
import jax.numpy as jnp
from jax import lax
from jax.experimental import pallas as pl
from jax.experimental.pallas import tpu as pltpu
from jax.experimental.pallas import tpu_sc as plsc

VOCAB = 1000000
SEQ_LEN = 200
EMBED = 32
BATCH = 4096

NC, NS = 2, 16
NW = NC * NS
N = BATCH * SEQ_LEN
ROWS_PER_W = N // NW
CHUNK = 2560
SUB = 128
NSUB = CHUNK // SUB
NCHUNKS = ROWS_PER_W // CHUNK


def _body(tok_hbm, idx_hbm, pos_hbm, out_hbm, idx_v, rows_v, pos_v, sem):
    wid = lax.axis_index("s") * NC + lax.axis_index("c")
    base_row = wid * ROWS_PER_W

    pltpu.sync_copy(pos_hbm, pos_v)

    def chunk_body(ci, _):
        row0 = base_row + ci * CHUNK
        irow0 = row0 // SUB
        pltpu.sync_copy(idx_hbm.at[pl.ds(irow0, NSUB)], idx_v)
        for j in range(NSUB):
            pltpu.async_copy(
                tok_hbm.at[idx_v.at[j]], rows_v.at[pl.ds(j * SUB, SUB)], sem
            )
        for j in range(NSUB):
            pltpu.make_async_copy(
                tok_hbm.at[idx_v.at[j]], rows_v.at[pl.ds(j * SUB, SUB)], sem
            ).wait()

        phase0 = lax.rem(row0, SEQ_LEN)

        def add_body(r, p):
            rows_v[r, 0:16] = rows_v[r, 0:16] + pos_v[p, 0:16]
            rows_v[r, 16:32] = rows_v[r, 16:32] + pos_v[p, 16:32]
            p = p + 1
            return jnp.where(p == SEQ_LEN, 0, p)

        lax.fori_loop(0, CHUNK, add_body, phase0)

        pltpu.sync_copy(rows_v, out_hbm.at[pl.ds(row0, CHUNK)])
        return 0

    lax.fori_loop(0, NCHUNKS, chunk_body, 0)


@jax.jit
def _run(tok, idx2d, pos):
    mesh = plsc.VectorSubcoreMesh(
        core_axis_name="c", subcore_axis_name="s", num_cores=NC, num_subcores=NS
    )
    return pl.kernel(
        _body,
        out_type=jax.ShapeDtypeStruct((N, EMBED), jnp.float32),
        mesh=mesh,
        scratch_types=[
            pltpu.VMEM((NSUB, SUB), jnp.int32),
            pltpu.VMEM((CHUNK, EMBED), jnp.float32),
            pltpu.VMEM((SEQ_LEN, EMBED), jnp.float32),
            pltpu.SemaphoreType.DMA,
        ],
    )(tok, idx2d, pos)


def kernel(inputs, token_table, pos_table):
    idx2d = inputs.astype(jnp.int32).reshape(N // SUB, SUB)
    out = _run(token_table, idx2d, pos_table)
    return out.reshape(BATCH, SEQ_LEN, EMBED)


# speedup vs baseline: 1.1722x; 1.1722x over previous
import functools

import jax
import jax.numpy as jnp
from jax import lax
from jax.experimental import pallas as pl
from jax.experimental.pallas import tpu as pltpu
from jax.experimental.pallas import tpu_sc as plsc

VOCAB = 1000000
SEQ_LEN = 200
EMBED = 32
BATCH = 4096

NC, NS = 2, 16
NW = NC * NS
N = BATCH * SEQ_LEN
ROWS_PER_W = N // NW
CHUNK = 1024
SUB = 128
NSUB = CHUNK // SUB
NCHUNKS = ROWS_PER_W // CHUNK


def _body(tok_hbm, idx_hbm, pos_hbm, out_hbm, idx_v, rows_v, pos_v, sem):
    wid = lax.axis_index("s") * NC + lax.axis_index("c")
    base_row = wid * ROWS_PER_W

    pltpu.sync_copy(pos_hbm, pos_v)

    def chunk_body(ci, _):
        row0 = pl.multiple_of(base_row + ci * CHUNK, CHUNK)
        irow0 = pl.multiple_of(row0 // SUB, NSUB)
        pltpu.sync_copy(idx_hbm.at[pl.ds(irow0, NSUB)], idx_v)
        for j in range(NSUB):
            pltpu.async_copy(
                tok_hbm.at[idx_v.at[j]], rows_v.at[pl.ds(j * SUB, SUB)], sem
            )
        for j in range(NSUB):
            pltpu.make_async_copy(
                tok_hbm.at[idx_v.at[j]], rows_v.at[pl.ds(j * SUB, SUB)], sem
            ).wait()

        phase0 = lax.rem(row0, SEQ_LEN)

        def add_body(r, p):
            rows_v[r, 0:16] = rows_v[r, 0:16] + pos_v[p, 0:16]
            rows_v[r, 16:32] = rows_v[r, 16:32] + pos_v[p, 16:32]
            p = p + 1
            return jnp.where(p == SEQ_LEN, 0, p)

        lax.fori_loop(0, CHUNK, add_body, phase0)

        pltpu.sync_copy(rows_v, out_hbm.at[pl.ds(row0, CHUNK)])
        return 0

    lax.fori_loop(0, NCHUNKS, chunk_body, 0)


@jax.jit
def _run(tok, idx2d, pos):
    mesh = plsc.VectorSubcoreMesh(
        core_axis_name="c", subcore_axis_name="s", num_cores=NC, num_subcores=NS
    )
    return pl.kernel(
        _body,
        out_type=jax.ShapeDtypeStruct((N, EMBED), jnp.float32),
        mesh=mesh,
        scratch_types=[
            pltpu.VMEM((NSUB, SUB), jnp.int32),
            pltpu.VMEM((CHUNK, EMBED), jnp.float32),
            pltpu.VMEM((SEQ_LEN, EMBED), jnp.float32),
            pltpu.SemaphoreType.DMA,
        ],
        compiler_params=pltpu.CompilerParams(use_tc_tiling_on_sc=False),
    )(tok, idx2d, pos)


def kernel(inputs, token_table, pos_table):
    idx2d = inputs.astype(jnp.int32).reshape(N // SUB, SUB)
    out = _run(token_table, idx2d, pos_table)
    return out.reshape(BATCH, SEQ_LEN, EMBED)


# speedup vs baseline: 1.4273x; 1.2176x over previous
import jax
import jax.numpy as jnp
from jax import lax
from jax.experimental import pallas as pl
from jax.experimental.pallas import tpu as pltpu
from jax.experimental.pallas import tpu_sc as plsc

VOCAB = 1000000
SEQ_LEN = 200
EMBED = 32
BATCH = 4096

NC, NS = 2, 16
NW = NC * NS
B_PER_W = BATCH // NW
CB = 8
NCHUNKS = B_PER_W // CB
SPLIT = 128


def _body(tok_hbm, idx_hbm, pos_hbm, out_hbm, idx_v, rows_v, pos_v, sem):
    wid = lax.axis_index("s") * NC + lax.axis_index("c")
    base_b = wid * B_PER_W

    pltpu.sync_copy(pos_hbm, pos_v)

    def chunk_body(ci, _):
        b0 = pl.multiple_of(base_b + ci * CB, CB)
        pltpu.sync_copy(idx_hbm.at[pl.ds(b0, CB)], idx_v)
        for b in range(CB):
            pltpu.async_copy(
                tok_hbm.at[idx_v.at[b, pl.ds(0, SPLIT)]],
                rows_v.at[b, pl.ds(0, SPLIT)],
                sem,
            )
            pltpu.async_copy(
                tok_hbm.at[idx_v.at[b, pl.ds(SPLIT, SEQ_LEN - SPLIT)]],
                rows_v.at[b, pl.ds(SPLIT, SEQ_LEN - SPLIT)],
                sem,
            )
        for b in range(CB):
            pltpu.make_async_copy(
                tok_hbm.at[idx_v.at[b, pl.ds(0, SPLIT)]],
                rows_v.at[b, pl.ds(0, SPLIT)],
                sem,
            ).wait()
            pltpu.make_async_copy(
                tok_hbm.at[idx_v.at[b, pl.ds(SPLIT, SEQ_LEN - SPLIT)]],
                rows_v.at[b, pl.ds(SPLIT, SEQ_LEN - SPLIT)],
                sem,
            ).wait()

        def add_body(l, _):
            p0 = pos_v[l, 0:16]
            p1 = pos_v[l, 16:32]
            for b in range(CB):
                rows_v[b, l, 0:16] = rows_v[b, l, 0:16] + p0
                rows_v[b, l, 16:32] = rows_v[b, l, 16:32] + p1
            return 0

        lax.fori_loop(0, SEQ_LEN, add_body, 0)

        pltpu.sync_copy(rows_v, out_hbm.at[pl.ds(b0, CB)])
        return 0

    lax.fori_loop(0, NCHUNKS, chunk_body, 0)


@jax.jit
def _run(tok, idx, pos):
    mesh = plsc.VectorSubcoreMesh(
        core_axis_name="c", subcore_axis_name="s", num_cores=NC, num_subcores=NS
    )
    return pl.kernel(
        _body,
        out_type=jax.ShapeDtypeStruct((BATCH, SEQ_LEN, EMBED), jnp.float32),
        mesh=mesh,
        scratch_types=[
            pltpu.VMEM((CB, SEQ_LEN), jnp.int32),
            pltpu.VMEM((CB, SEQ_LEN, EMBED), jnp.float32),
            pltpu.VMEM((SEQ_LEN, EMBED), jnp.float32),
            pltpu.SemaphoreType.DMA,
        ],
        compiler_params=pltpu.CompilerParams(use_tc_tiling_on_sc=False),
    )(tok, idx, pos)


def kernel(inputs, token_table, pos_table):
    return _run(token_table, inputs.astype(jnp.int32), pos_table)
